# baseline (speedup 1.0000x reference)
import functools

import jax
import jax.numpy as jnp
from jax import lax
from jax.experimental import pallas as pl
from jax.experimental.pallas import tpu as pltpu
from jax.experimental.pallas import tpu_sc as plsc

_NEG_INF = -10.0 ** 10
_B = 4096
_V = 100000
_NUM_CORES = 2
_NUM_SUBCORES = 16
_NW = _NUM_CORES * _NUM_SUBCORES
_RPW = _B // _NW
_NCHUNK = 5
_CHUNK = _V // _NCHUNK
_LANES = 16
_ACCS = 10
_STEP = _LANES * _ACCS
_GATHER_1D = lax.GatherDimensionNumbers(
    offset_dims=(), collapsed_slice_dims=(0,), start_index_map=(0,))


def _permute(x, idx):
    return lax.gather(
        x, idx.reshape(_LANES, 1), dimension_numbers=_GATHER_1D,
        slice_sizes=(1,), mode=lax.GatherScatterMode.PROMISE_IN_BOUNDS)


def _xlane_max(x, lane):
    for s in (8, 4, 2, 1):
        x = jnp.maximum(x, _permute(x, lane ^ s))
    return x


def _sc_body(pred_ref, label_ref, out_ref, label_v, b0, b1, b2, b3,
             b4, sums_v, sems):
    bufs = (b0, b1, b2, b3, b4)
    wid = lax.axis_index("s") * _NUM_CORES + lax.axis_index("c")
    row0 = wid * _RPW

    pltpu.sync_copy(label_ref.at[pl.ds(row0, _RPW)], label_v)
    sums_v[...] = jnp.zeros((_LANES,), jnp.float32)

    for b in range(_NCHUNK):
        pltpu.async_copy(
            pred_ref.at[pl.ds(row0 * _V + b * _CHUNK, _CHUNK)],
            bufs[b], sems.at[b])

    def group_body(jg, carry):
        lane = lax.broadcasted_iota(jnp.int32, (_LANES,), 0)
        negv = jnp.full((_LANES,), _NEG_INF, jnp.float32)
        lbl16 = label_v[pl.ds(jg * _LANES, _LANES)]
        hsum = jnp.zeros((_LANES,), jnp.float32)
        for jj in range(_LANES):
            j = jg * _LANES + jj
            lbl_s = lbl16[jj]
            lblv = jnp.full((_LANES,), lbl_s, jnp.int32)
            fyv = negv
            accs = [negv] * _ACCS
            for b in range(_NCHUNK):
                pltpu.make_async_copy(
                    pred_ref.at[pl.ds(0, _CHUNK)], bufs[b],
                    sems.at[b]).wait()
                buf = bufs[b]
                local_s = lbl_s - (b * _CHUNK)
                base = jnp.minimum(
                    jnp.maximum((local_s >> 4) << 4, 0), _CHUNK - _LANES)
                glob = lane + (base + b * _CHUNK)
                is_lbl = glob == lblv
                v16 = buf[pl.ds(base, _LANES)]
                fyv = jnp.maximum(fyv, jnp.where(is_lbl, v16, negv))
                buf[pl.ds(base, _LANES)] = jnp.where(is_lbl, negv, v16)

                @plsc.parallel_loop(0, _CHUNK, _STEP, carry=tuple(accs))
                def chunk_red(k, acc):
                    return tuple(
                        jnp.maximum(a, buf[pl.ds(k + u * _LANES, _LANES)])
                        for u, a in enumerate(acc))

                accs = list(chunk_red)

                @pl.when(j < _RPW - 1)
                def _():
                    pltpu.async_copy(
                        pred_ref.at[
                            pl.ds((row0 + j + 1) * _V + b * _CHUNK, _CHUNK)],
                        bufs[b], sems.at[b])

            m = accs[0]
            for a in accs[1:]:
                m = jnp.maximum(m, a)
            fnym = _xlane_max(m, lane)
            fy = _xlane_max(fyv, lane)
            hsum = hsum + jnp.where(
                lane == 0,
                jnp.maximum(2.0 - fy, 0.0) + jnp.maximum(1.0 + fnym, 0.0),
                0.0)
        sums_v[...] = sums_v[...] + hsum
        return carry

    lax.fori_loop(0, _RPW // _LANES, group_body, 0)
    pltpu.sync_copy(sums_v, out_ref.at[wid])


@jax.jit
def _sc_loss(pred_flat, lbl):
    mesh = plsc.VectorSubcoreMesh(
        core_axis_name="c", subcore_axis_name="s",
        num_cores=_NUM_CORES, num_subcores=_NUM_SUBCORES)
    part = pl.kernel(
        _sc_body,
        out_type=jax.ShapeDtypeStruct((_NW, _LANES), jnp.float32),
        mesh=mesh,
        scratch_types=[
            pltpu.VMEM((_RPW,), jnp.int32),
            pltpu.VMEM((_CHUNK,), jnp.float32),
            pltpu.VMEM((_CHUNK,), jnp.float32),
            pltpu.VMEM((_CHUNK,), jnp.float32),
            pltpu.VMEM((_CHUNK,), jnp.float32),
            pltpu.VMEM((_CHUNK,), jnp.float32),
            pltpu.VMEM((_LANES,), jnp.float32),
            pltpu.SemaphoreType.DMA((_NCHUNK,)),
        ],
    )(pred_flat, lbl)
    return jnp.sum(part) * (1.0 / _B)


def kernel(prediction, label):
    return _sc_loss(prediction.reshape(-1), label.astype(jnp.int32))


# speedup vs baseline: 1.9540x; 1.9540x over previous
import jax
import jax.numpy as jnp
from jax import lax
from jax.experimental import pallas as pl
from jax.experimental.pallas import tpu as pltpu
from jax.experimental.pallas import tpu_sc as plsc

_NEG_INF = -10.0 ** 10
_B = 4096
_V = 100000
_SUB = 8
_TILE = 128
_LANES = 16
_NUM_CORES = 2
_NUM_SUBCORES = 16
_NW = _NUM_CORES * _NUM_SUBCORES
_BANDS = _B // _SUB
_BPW = _BANDS // _NW
_MAIN_TILES = _V // _TILE
_TAIL = _V - _MAIN_TILES * _TILE
_SLOT_TILES = 16
_SLOT_COLS = _SLOT_TILES * _TILE
_NFULL = _MAIN_TILES // _SLOT_TILES
_PART_TILES = _MAIN_TILES - _NFULL * _SLOT_TILES
_PART_COLS = _PART_TILES * _TILE
_PART_COL0 = _NFULL * _SLOT_COLS
_NSLOT = _NFULL + 1
_NBUF = 7
_NROUND = _NSLOT // _NBUF - 1

_GATHER_1D = lax.GatherDimensionNumbers(
    offset_dims=(), collapsed_slice_dims=(0,), start_index_map=(0,))


def _permute(x, idx):
    return lax.gather(
        x, idx.reshape(_LANES, 1), dimension_numbers=_GATHER_1D,
        slice_sizes=(1,), mode=lax.GatherScatterMode.PROMISE_IN_BOUNDS)


def _xlane_max(x, lane):
    for s in (8, 4, 2, 1):
        x = jnp.maximum(x, _permute(x, lane ^ s))
    return x


def _sc_body(pred_ref, tail_ref, label_ref, out_ref, label_v,
             f0, f1, f2, f3, f4, f5, f6, tbuf, sums_v, sems, tsem):
    bufs = (f0, f1, f2, f3, f4, f5, f6)
    wid = lax.axis_index("s") * _NUM_CORES + lax.axis_index("c")
    band0 = wid * _BPW

    pltpu.sync_copy(label_ref.at[pl.ds(band0 * _SUB, _BPW * _SUB)],
                    label_v.at[pl.ds(0, _BPW * _SUB)])
    sums_v[...] = jnp.zeros((_LANES,), jnp.float32)

    def rows_of(bd_g):
        return pl.ds(pl.multiple_of(bd_g * _SUB, _SUB), _SUB)

    def start_full(bd_g, col0, u):
        pltpu.async_copy(
            pred_ref.at[rows_of(bd_g),
                        pl.ds(pl.multiple_of(col0, _SLOT_COLS), _SLOT_COLS)],
            bufs[u], sems.at[u])

    def start_part(bd_g):
        pltpu.async_copy(
            pred_ref.at[rows_of(bd_g), pl.ds(_PART_COL0, _PART_COLS)],
            bufs[_NBUF - 1].at[:, pl.ds(0, _PART_COLS)],
            sems.at[_NBUF - 1])

    def wait_full(u):
        pltpu.make_async_copy(
            pred_ref.at[pl.ds(0, _SUB), pl.ds(0, _SLOT_COLS)],
            bufs[u], sems.at[u]).wait()

    def wait_part():
        pltpu.make_async_copy(
            pred_ref.at[pl.ds(0, _SUB), pl.ds(0, _PART_COLS)],
            bufs[_NBUF - 1].at[:, pl.ds(0, _PART_COLS)],
            sems.at[_NBUF - 1]).wait()

    def start_tail(bd_g):
        pltpu.async_copy(
            tail_ref.at[pl.ds(pl.multiple_of(bd_g * _SUB * _TAIL,
                                             _SUB * _TAIL), _SUB * _TAIL)],
            tbuf, tsem)

    for u in range(_NBUF):
        start_full(band0, u * _SLOT_COLS, u)
    start_tail(band0)

    def mask_fy(buf, r, lbl_r, col0, width, lane, negv, fy_r):
        local = lbl_r - col0
        base = pl.multiple_of(
            jnp.minimum(jnp.maximum((local >> 4) << 4, 0), width - _LANES),
            _LANES)
        v16 = buf[r, pl.ds(base, _LANES)]
        isl = (lane + (col0 + base)) == jnp.full((_LANES,), lbl_r, jnp.int32)
        fy_r = jnp.maximum(fy_r, jnp.where(isl, v16, negv))
        buf[r, pl.ds(base, _LANES)] = jnp.where(isl, negv, v16)
        return fy_r

    def reduce_slot(buf, ntiles, accs):
        @plsc.parallel_loop(0, ntiles, 1, carry=tuple(accs))
        def red(t, acc):
            out = []
            for r in range(_SUB):
                a = acc[r]
                for v in range(_TILE // _LANES):
                    a = jnp.maximum(
                        a, buf[r, pl.ds(t * _TILE + v * _LANES, _LANES)])
                out.append(a)
            return tuple(out)
        return list(red)

    def band_body(bd, carry):
        bd_g = band0 + bd
        lane = lax.broadcasted_iota(jnp.int32, (_LANES,), 0)
        negv = jnp.full((_LANES,), _NEG_INF, jnp.float32)
        lbl8 = label_v[pl.ds(pl.multiple_of(bd * _SUB, _SUB), _LANES)]
        lbls = [lbl8[r] for r in range(_SUB)]

        def round_body(ri, rcarry):
            accs = list(rcarry[:_SUB])
            fys = list(rcarry[_SUB:])
            lane_i = lax.broadcasted_iota(jnp.int32, (_LANES,), 0)
            negv_i = jnp.full((_LANES,), _NEG_INF, jnp.float32)
            for u in range(_NBUF):
                s = ri * _NBUF + u
                col0 = s * _SLOT_COLS
                wait_full(u)
                for r in range(_SUB):
                    fys[r] = mask_fy(bufs[u], r, lbls[r], col0, _SLOT_COLS,
                                     lane_i, negv_i, fys[r])
                accs = reduce_slot(bufs[u], _SLOT_TILES, accs)
                if u < _NBUF - 1:
                    start_full(bd_g, col0 + _NBUF * _SLOT_COLS, u)
                else:
                    @pl.when(ri < _NROUND - 1)
                    def _():
                        start_full(bd_g, col0 + _NBUF * _SLOT_COLS, u)

                    @pl.when(ri == _NROUND - 1)
                    def _():
                        start_part(bd_g)
            return tuple(accs) + tuple(fys)

        init = (jnp.full((_LANES,), _NEG_INF, jnp.float32),) * (2 * _SUB)
        rres = lax.fori_loop(0, _NROUND, round_body, init)
        accs = list(rres[:_SUB])
        fys = list(rres[_SUB:])

        for u in range(_NBUF - 1):
            s = _NROUND * _NBUF + u
            col0 = s * _SLOT_COLS
            wait_full(u)
            for r in range(_SUB):
                fys[r] = mask_fy(bufs[u], r, lbls[r], col0, _SLOT_COLS,
                                 lane, negv, fys[r])
            accs = reduce_slot(bufs[u], _SLOT_TILES, accs)

            @pl.when(bd < _BPW - 1)
            def _():
                start_full(bd_g + 1, u * _SLOT_COLS, u)

        wait_part()
        for r in range(_SUB):
            fys[r] = mask_fy(bufs[_NBUF - 1], r, lbls[r], _PART_COL0,
                             _PART_COLS, lane, negv, fys[r])
        accs = reduce_slot(bufs[_NBUF - 1], _PART_TILES, accs)

        @pl.when(bd < _BPW - 1)
        def _():
            start_full(bd_g + 1, (_NBUF - 1) * _SLOT_COLS, _NBUF - 1)

        pltpu.make_async_copy(
            tail_ref.at[pl.ds(0, _SUB * _TAIL)], tbuf, tsem).wait()
        for r in range(_SUB):
            lblv = jnp.full((_LANES,), lbls[r], jnp.int32)
            for v in range(_TAIL // _LANES):
                x = tbuf[pl.ds(r * _TAIL + v * _LANES, _LANES)]
                isl = (lane + (_MAIN_TILES * _TILE + v * _LANES)) == lblv
                fys[r] = jnp.maximum(fys[r], jnp.where(isl, x, negv))
                accs[r] = jnp.maximum(accs[r], jnp.where(isl, negv, x))

        @pl.when(bd < _BPW - 1)
        def _():
            start_tail(bd_g + 1)

        hsum = jnp.zeros((_LANES,), jnp.float32)
        for r in range(_SUB):
            fnym = _xlane_max(accs[r], lane)
            fy = _xlane_max(fys[r], lane)
            hsum = hsum + jnp.where(
                lane == 0,
                jnp.maximum(2.0 - fy, 0.0) + jnp.maximum(1.0 + fnym, 0.0),
                0.0)
        sums_v[...] = sums_v[...] + hsum
        return carry

    lax.fori_loop(0, _BPW, band_body, 0)
    pltpu.sync_copy(sums_v, out_ref.at[wid])


@jax.jit
def _sc_loss(pred, tail, lbl):
    mesh = plsc.VectorSubcoreMesh(
        core_axis_name="c", subcore_axis_name="s",
        num_cores=_NUM_CORES, num_subcores=_NUM_SUBCORES)
    part = pl.kernel(
        _sc_body,
        out_type=jax.ShapeDtypeStruct((_NW, _LANES), jnp.float32),
        mesh=mesh,
        compiler_params=pltpu.CompilerParams(use_tc_tiling_on_sc=True),
        scratch_types=(
            [pltpu.VMEM((_BPW * _SUB + _LANES,), jnp.int32)]
            + [pltpu.VMEM((_SUB, _SLOT_COLS), jnp.float32)] * _NBUF
            + [pltpu.VMEM((_SUB * _TAIL,), jnp.float32),
               pltpu.VMEM((_LANES,), jnp.float32),
               pltpu.SemaphoreType.DMA((_NBUF,)),
               pltpu.SemaphoreType.DMA]
        ),
    )(pred, tail, lbl)
    return jnp.sum(part) * (1.0 / _B)


def kernel(prediction, label):
    tail = prediction[:, _MAIN_TILES * _TILE:].reshape(-1)
    return _sc_loss(prediction, tail, label.astype(jnp.int32))


# speedup vs baseline: 1.9571x; 1.0016x over previous
import jax
import jax.numpy as jnp
from jax import lax
from jax.experimental import pallas as pl
from jax.experimental.pallas import tpu as pltpu
from jax.experimental.pallas import tpu_sc as plsc

_NEG_INF = -10.0 ** 10
_B = 4096
_V = 100000
_SUB = 8
_TILE = 128
_LANES = 16
_NUM_CORES = 2
_NUM_SUBCORES = 16
_NW = _NUM_CORES * _NUM_SUBCORES
_BANDS = _B // _SUB
_BPW = _BANDS // _NW
_MAIN_TILES = _V // _TILE
_TAIL = _V - _MAIN_TILES * _TILE
_TAIL0 = _MAIN_TILES * _TILE
_SLOT_TILES = 16
_SLOT_COLS = _SLOT_TILES * _TILE
_NFULL = _MAIN_TILES // _SLOT_TILES
_PART_TILES = _MAIN_TILES - _NFULL * _SLOT_TILES
_PART_COLS = _PART_TILES * _TILE
_PART_COL0 = _NFULL * _SLOT_COLS
_NSLOT = _NFULL + 1
_NBUF = 7
_NROUND = _NSLOT // _NBUF - 1

_GATHER_1D = lax.GatherDimensionNumbers(
    offset_dims=(), collapsed_slice_dims=(0,), start_index_map=(0,))


def _permute(x, idx):
    return lax.gather(
        x, idx.reshape(_LANES, 1), dimension_numbers=_GATHER_1D,
        slice_sizes=(1,), mode=lax.GatherScatterMode.PROMISE_IN_BOUNDS)


def _xlane_max(x, lane):
    for s in (8, 4, 2, 1):
        x = jnp.maximum(x, _permute(x, lane ^ s))
    return x


def _tc_tail_body(pred_ref, lbl_ref, tmax_ref, tfy_ref):
    x = pred_ref[...]
    lbl = lbl_ref[...]
    cols = lax.broadcasted_iota(jnp.int32, (_B, _TILE), 1) + _TAIL0
    valid = cols < _V
    isl = cols == lbl
    tmax_ref[...] = jnp.max(
        jnp.where(valid & jnp.logical_not(isl), x, _NEG_INF), axis=1)
    tfy_ref[...] = jnp.max(jnp.where(isl, x, _NEG_INF), axis=1)


def _sc_body(pred_ref, label_ref, tmax_ref, tfy_ref, out_ref, label_v,
             tmax_v, tfy_v, f0, f1, f2, f3, f4, f5, f6, sums_v, sems):
    bufs = (f0, f1, f2, f3, f4, f5, f6)
    wid = lax.axis_index("s") * _NUM_CORES + lax.axis_index("c")
    band0 = wid * _BPW
    nrows = _BPW * _SUB

    pltpu.sync_copy(label_ref.at[pl.ds(band0 * _SUB, nrows)],
                    label_v.at[pl.ds(0, nrows)])
    pltpu.sync_copy(tmax_ref.at[pl.ds(band0 * _SUB, nrows)],
                    tmax_v.at[pl.ds(0, nrows)])
    pltpu.sync_copy(tfy_ref.at[pl.ds(band0 * _SUB, nrows)],
                    tfy_v.at[pl.ds(0, nrows)])
    sums_v[...] = jnp.zeros((_LANES,), jnp.float32)

    def rows_of(bd_g):
        return pl.ds(pl.multiple_of(bd_g * _SUB, _SUB), _SUB)

    def start_full(bd_g, col0, u):
        pltpu.async_copy(
            pred_ref.at[rows_of(bd_g),
                        pl.ds(pl.multiple_of(col0, _SLOT_COLS), _SLOT_COLS)],
            bufs[u], sems.at[u])

    def start_part(bd_g):
        pltpu.async_copy(
            pred_ref.at[rows_of(bd_g), pl.ds(_PART_COL0, _PART_COLS)],
            bufs[_NBUF - 1].at[:, pl.ds(0, _PART_COLS)],
            sems.at[_NBUF - 1])

    def wait_full(u):
        pltpu.make_async_copy(
            pred_ref.at[pl.ds(0, _SUB), pl.ds(0, _SLOT_COLS)],
            bufs[u], sems.at[u]).wait()

    def wait_part():
        pltpu.make_async_copy(
            pred_ref.at[pl.ds(0, _SUB), pl.ds(0, _PART_COLS)],
            bufs[_NBUF - 1].at[:, pl.ds(0, _PART_COLS)],
            sems.at[_NBUF - 1]).wait()

    for u in range(_NBUF):
        start_full(band0, u * _SLOT_COLS, u)

    def mask_fy(buf, r, lbl_r, col0, width, lane, negv, fy_r):
        local = lbl_r - col0
        base = pl.multiple_of(
            jnp.minimum(jnp.maximum((local >> 4) << 4, 0), width - _LANES),
            _LANES)
        v16 = buf[r, pl.ds(base, _LANES)]
        isl = (lane + (col0 + base)) == jnp.full((_LANES,), lbl_r, jnp.int32)
        fy_r = jnp.maximum(fy_r, jnp.where(isl, v16, negv))
        buf[r, pl.ds(base, _LANES)] = jnp.where(isl, negv, v16)
        return fy_r

    def reduce_slot(buf, ntiles, accs):
        @plsc.parallel_loop(0, ntiles, 1, carry=tuple(accs))
        def red(t, acc):
            out = []
            for r in range(_SUB):
                a = acc[r]
                for v in range(_TILE // _LANES):
                    a = jnp.maximum(
                        a, buf[r, pl.ds(t * _TILE + v * _LANES, _LANES)])
                out.append(a)
            return tuple(out)
        return list(red)

    def band_body(bd, carry):
        bd_g = band0 + bd
        lane = lax.broadcasted_iota(jnp.int32, (_LANES,), 0)
        negv = jnp.full((_LANES,), _NEG_INF, jnp.float32)
        lbl8 = label_v[pl.ds(pl.multiple_of(bd * _SUB, _SUB), _LANES)]
        tm8 = tmax_v[pl.ds(pl.multiple_of(bd * _SUB, _SUB), _LANES)]
        tf8 = tfy_v[pl.ds(pl.multiple_of(bd * _SUB, _SUB), _LANES)]
        lbls = [lbl8[r] for r in range(_SUB)]

        def round_body(ri, rcarry):
            accs = list(rcarry[:_SUB])
            fys = list(rcarry[_SUB:])
            lane_i = lax.broadcasted_iota(jnp.int32, (_LANES,), 0)
            negv_i = jnp.full((_LANES,), _NEG_INF, jnp.float32)
            for u in range(_NBUF):
                s = ri * _NBUF + u
                col0 = s * _SLOT_COLS
                wait_full(u)
                for r in range(_SUB):
                    fys[r] = mask_fy(bufs[u], r, lbls[r], col0, _SLOT_COLS,
                                     lane_i, negv_i, fys[r])
                accs = reduce_slot(bufs[u], _SLOT_TILES, accs)
                if u < _NBUF - 1:
                    start_full(bd_g, col0 + _NBUF * _SLOT_COLS, u)
                else:
                    @pl.when(ri < _NROUND - 1)
                    def _():
                        start_full(bd_g, col0 + _NBUF * _SLOT_COLS, u)

                    @pl.when(ri == _NROUND - 1)
                    def _():
                        start_part(bd_g)
            return tuple(accs) + tuple(fys)

        init = (jnp.full((_LANES,), _NEG_INF, jnp.float32),) * (2 * _SUB)
        rres = lax.fori_loop(0, _NROUND, round_body, init)
        accs = list(rres[:_SUB])
        fys = list(rres[_SUB:])

        for u in range(_NBUF - 1):
            s = _NROUND * _NBUF + u
            col0 = s * _SLOT_COLS
            wait_full(u)
            for r in range(_SUB):
                fys[r] = mask_fy(bufs[u], r, lbls[r], col0, _SLOT_COLS,
                                 lane, negv, fys[r])
            accs = reduce_slot(bufs[u], _SLOT_TILES, accs)

            @pl.when(bd < _BPW - 1)
            def _():
                start_full(bd_g + 1, u * _SLOT_COLS, u)

        wait_part()
        for r in range(_SUB):
            fys[r] = mask_fy(bufs[_NBUF - 1], r, lbls[r], _PART_COL0,
                             _PART_COLS, lane, negv, fys[r])
        accs = reduce_slot(bufs[_NBUF - 1], _PART_TILES, accs)

        @pl.when(bd < _BPW - 1)
        def _():
            start_full(bd_g + 1, (_NBUF - 1) * _SLOT_COLS, _NBUF - 1)

        hsum = jnp.zeros((_LANES,), jnp.float32)
        for r in range(_SUB):
            a = jnp.maximum(accs[r], jnp.full((_LANES,), tm8[r], jnp.float32))
            f = jnp.maximum(fys[r], jnp.full((_LANES,), tf8[r], jnp.float32))
            fnym = _xlane_max(a, lane)
            fy = _xlane_max(f, lane)
            hsum = hsum + jnp.where(
                lane == 0,
                jnp.maximum(2.0 - fy, 0.0) + jnp.maximum(1.0 + fnym, 0.0),
                0.0)
        sums_v[...] = sums_v[...] + hsum
        return carry

    lax.fori_loop(0, _BPW, band_body, 0)
    pltpu.sync_copy(sums_v, out_ref.at[wid])


@jax.jit
def _sc_loss(pred, lbl):
    tmax, tfy = pl.pallas_call(
        _tc_tail_body,
        grid=(1,),
        in_specs=[
            pl.BlockSpec((_B, _TILE), lambda i: (0, _TAIL0 // _TILE)),
            pl.BlockSpec((_B, 1), lambda i: (0, 0)),
        ],
        out_specs=[
            pl.BlockSpec((_B,), lambda i: (0,)),
            pl.BlockSpec((_B,), lambda i: (0,)),
        ],
        out_shape=[
            jax.ShapeDtypeStruct((_B,), jnp.float32),
            jax.ShapeDtypeStruct((_B,), jnp.float32),
        ],
    )(pred, lbl.reshape(_B, 1))

    mesh = plsc.VectorSubcoreMesh(
        core_axis_name="c", subcore_axis_name="s",
        num_cores=_NUM_CORES, num_subcores=_NUM_SUBCORES)
    part = pl.kernel(
        _sc_body,
        out_type=jax.ShapeDtypeStruct((_NW, _LANES), jnp.float32),
        mesh=mesh,
        compiler_params=pltpu.CompilerParams(use_tc_tiling_on_sc=True),
        scratch_types=(
            [pltpu.VMEM((_BPW * _SUB + _LANES,), jnp.int32),
             pltpu.VMEM((_BPW * _SUB + _LANES,), jnp.float32),
             pltpu.VMEM((_BPW * _SUB + _LANES,), jnp.float32)]
            + [pltpu.VMEM((_SUB, _SLOT_COLS), jnp.float32)] * _NBUF
            + [pltpu.VMEM((_LANES,), jnp.float32),
               pltpu.SemaphoreType.DMA((_NBUF,))]
        ),
    )(pred, lbl, tmax, tfy)
    return jnp.sum(part) * (1.0 / _B)


def kernel(prediction, label):
    return _sc_loss(prediction, label.astype(jnp.int32))


# speedup vs baseline: 2.6819x; 1.3703x over previous
import jax
import jax.numpy as jnp
from jax import lax
from jax.experimental import pallas as pl
from jax.experimental.pallas import tpu as pltpu
from jax.experimental.pallas import tpu_sc as plsc

_NEG_INF = -10.0 ** 10
_B = 4096
_V = 100000
_SUB = 8
_LANES = 16
_GROUPS = 8
_WCOLS = 128
_NUM_CORES = 2
_NUM_SUBCORES = 16
_NW = _NUM_CORES * _NUM_SUBCORES
_NB_SLOT = 25
_SLOT_ROWS = _NB_SLOT * _SUB
_NSLOT = _V // _SLOT_ROWS
_NBUF = 4
_NROUND = _NSLOT // _NBUF

_GATHER_1D = lax.GatherDimensionNumbers(
    offset_dims=(), collapsed_slice_dims=(0,), start_index_map=(0,))


def _permute(x, idx):
    return lax.gather(
        x, idx.reshape(_LANES, 1), dimension_numbers=_GATHER_1D,
        slice_sizes=(1,), mode=lax.GatherScatterMode.PROMISE_IN_BOUNDS)


def _xlane_max(x, lane):
    for s in (8, 4, 2, 1):
        x = jnp.maximum(x, _permute(x, lane ^ s))
    return x


def _sc_body(pred_ref, label_ref, out_ref, label_v, accb, fyb,
             b0, b1, b2, b3, sums_v, sems):
    bufs = (b0, b1, b2, b3)
    wid = lax.axis_index("s") * _NUM_CORES + lax.axis_index("c")
    col0 = pl.multiple_of(wid * _WCOLS, _WCOLS)

    pltpu.sync_copy(label_ref.at[pl.ds(col0, _WCOLS)], label_v)
    for g in range(_GROUPS):
        accb[pl.ds(g * _LANES, _LANES)] = jnp.full(
            (_LANES,), _NEG_INF, jnp.float32)
        fyb[pl.ds(g * _LANES, _LANES)] = jnp.full(
            (_LANES,), _NEG_INF, jnp.float32)

    def start_slot(s, u):
        pltpu.async_copy(
            pred_ref.at[pl.ds(pl.multiple_of(s * _SLOT_ROWS, _SUB),
                              _SLOT_ROWS),
                        pl.ds(col0, _WCOLS)],
            bufs[u], sems.at[u])

    def wait_slot(u):
        pltpu.make_async_copy(
            pred_ref.at[pl.ds(0, _SLOT_ROWS), pl.ds(0, _WCOLS)],
            bufs[u], sems.at[u]).wait()

    for u in range(_NBUF):
        start_slot(u, u)

    def process_slot(s, u):
        lane = lax.broadcasted_iota(jnp.int32, (_LANES,), 0)
        negv = jnp.full((_LANES,), _NEG_INF, jnp.float32)
        wait_slot(u)
        buf = bufs[u]
        base = s * _SLOT_ROWS
        lblv = [label_v[pl.ds(g * _LANES, _LANES)] for g in range(_GROUPS)]

        hitv = jnp.zeros((_LANES,), jnp.float32)
        for g in range(_GROUPS):
            loc = lblv[g] - base
            inb = (loc >= 0) & (loc < _SLOT_ROWS)
            hitv = hitv + jnp.where(inb, 1.0, 0.0)
        hit = _xlane_max(hitv, lane)[0] > 0.5

        @pl.when(jnp.logical_not(hit))
        def _():
            accs = [accb[pl.ds(g * _LANES, _LANES)] for g in range(_GROUPS)]

            @plsc.parallel_loop(0, _NB_SLOT, 1, carry=tuple(accs))
            def red(t, acc):
                out = list(acc)
                for r in range(_SUB):
                    row = t * _SUB + r
                    for g in range(_GROUPS):
                        out[g] = jnp.maximum(
                            out[g], buf[row, pl.ds(g * _LANES, _LANES)])
                return tuple(out)

            for g in range(_GROUPS):
                accb[pl.ds(g * _LANES, _LANES)] = red[g]

        @pl.when(hit)
        def _():
            accs = [accb[pl.ds(g * _LANES, _LANES)] for g in range(_GROUPS)]
            fys = [fyb[pl.ds(g * _LANES, _LANES)] for g in range(_GROUPS)]

            @plsc.parallel_loop(0, _NB_SLOT, 1,
                                carry=tuple(accs) + tuple(fys))
            def red(t, c):
                out = list(c[:_GROUPS])
                fy = list(c[_GROUPS:])
                for r in range(_SUB):
                    row = t * _SUB + r
                    rowv = jnp.full((_LANES,), base + row, jnp.int32)
                    for g in range(_GROUPS):
                        x = buf[row, pl.ds(g * _LANES, _LANES)]
                        isl = lblv[g] == rowv
                        out[g] = jnp.maximum(
                            out[g], jnp.where(isl, negv, x))
                        fy[g] = jnp.maximum(
                            fy[g], jnp.where(isl, x, negv))
                return tuple(out) + tuple(fy)

            for g in range(_GROUPS):
                accb[pl.ds(g * _LANES, _LANES)] = red[g]
                fyb[pl.ds(g * _LANES, _LANES)] = red[_GROUPS + g]

    def round_body(ri, carry):
        for u in range(_NBUF):
            s = ri * _NBUF + u
            process_slot(s, u)

            @pl.when(s < _NSLOT - _NBUF)
            def _():
                start_slot(s + _NBUF, u)
        return carry

    lax.fori_loop(0, _NROUND, round_body, 0)

    hsum = jnp.zeros((_LANES,), jnp.float32)
    for g in range(_GROUPS):
        fnym = accb[pl.ds(g * _LANES, _LANES)]
        fy = fyb[pl.ds(g * _LANES, _LANES)]
        hsum = hsum + (jnp.maximum(2.0 - fy, 0.0)
                       + jnp.maximum(1.0 + fnym, 0.0))
    sums_v[...] = hsum
    pltpu.sync_copy(sums_v, out_ref.at[wid])


@jax.jit
def _sc_loss(pred_t, lbl):
    mesh = plsc.VectorSubcoreMesh(
        core_axis_name="c", subcore_axis_name="s",
        num_cores=_NUM_CORES, num_subcores=_NUM_SUBCORES)
    part = pl.kernel(
        _sc_body,
        out_type=jax.ShapeDtypeStruct((_NW, _LANES), jnp.float32),
        mesh=mesh,
        compiler_params=pltpu.CompilerParams(use_tc_tiling_on_sc=True),
        scratch_types=(
            [pltpu.VMEM((_WCOLS,), jnp.int32),
             pltpu.VMEM((_WCOLS,), jnp.float32),
             pltpu.VMEM((_WCOLS,), jnp.float32)]
            + [pltpu.VMEM((_SLOT_ROWS, _WCOLS), jnp.float32)] * _NBUF
            + [pltpu.VMEM((_LANES,), jnp.float32),
               pltpu.SemaphoreType.DMA((_NBUF,))]
        ),
    )(pred_t, lbl)
    return jnp.sum(part) * (1.0 / _B)


def kernel(prediction, label):
    return _sc_loss(prediction.T, label.astype(jnp.int32))


# speedup vs baseline: 6.7460x; 2.5154x over previous
import jax
import jax.numpy as jnp
from jax import lax
from jax.experimental import pallas as pl
from jax.experimental.pallas import tpu as pltpu
from jax.experimental.pallas import tpu_sc as plsc

_NEG_INF = -10.0 ** 10
_B = 4096
_V = 100000
_SUB = 8
_LANES = 16
_GROUPS = 8
_WCOLS = 128
_NUM_CORES = 2
_NUM_SUBCORES = 16
_NW = _NUM_CORES * _NUM_SUBCORES
_NB_SLOT = 25
_SLOT_ROWS = _NB_SLOT * _SUB
_NSLOT = _V // _SLOT_ROWS
_NBUF = 4
_NROUND = _NSLOT // _NBUF

_GATHER_1D = lax.GatherDimensionNumbers(
    offset_dims=(), collapsed_slice_dims=(0,), start_index_map=(0,))


def _permute(x, idx):
    return lax.gather(
        x, idx.reshape(_LANES, 1), dimension_numbers=_GATHER_1D,
        slice_sizes=(1,), mode=lax.GatherScatterMode.PROMISE_IN_BOUNDS)


def _xlane_max(x, lane):
    for s in (8, 4, 2, 1):
        x = jnp.maximum(x, _permute(x, lane ^ s))
    return x


def _sc_body(pred_ref, label_ref, out_ref, label_v, accb, fyb,
             b0, b1, b2, b3, sums_v, sems):
    bufs = (b0, b1, b2, b3)
    wid = lax.axis_index("s") * _NUM_CORES + lax.axis_index("c")
    col0 = pl.multiple_of(wid * _WCOLS, _WCOLS)

    pltpu.sync_copy(label_ref.at[pl.ds(col0, _WCOLS)], label_v)
    for g in range(_GROUPS):
        accb[pl.ds(g * _LANES, _LANES)] = jnp.full(
            (_LANES,), _NEG_INF, jnp.float32)
        fyb[pl.ds(g * _LANES, _LANES)] = jnp.full(
            (_LANES,), _NEG_INF, jnp.float32)

    def start_slot(s, u):
        pltpu.async_copy(
            pred_ref.at[pl.ds(pl.multiple_of(s * _SLOT_ROWS, _SUB),
                              _SLOT_ROWS),
                        pl.ds(col0, _WCOLS)],
            bufs[u], sems.at[u])

    def wait_slot(u):
        pltpu.make_async_copy(
            pred_ref.at[pl.ds(0, _SLOT_ROWS), pl.ds(0, _WCOLS)],
            bufs[u], sems.at[u]).wait()

    for u in range(_NBUF):
        start_slot(u, u)

    def process_slot(s, u):
        lane = lax.broadcasted_iota(jnp.int32, (_LANES,), 0)
        negv = jnp.full((_LANES,), _NEG_INF, jnp.float32)
        wait_slot(u)
        buf = bufs[u]
        base = s * _SLOT_ROWS
        lblv = [label_v[pl.ds(g * _LANES, _LANES)] for g in range(_GROUPS)]

        hitv = jnp.zeros((_LANES,), jnp.float32)
        for g in range(_GROUPS):
            loc = lblv[g] - base
            inb = (loc >= 0) & (loc < _SLOT_ROWS)
            hitv = hitv + jnp.where(inb, 1.0, 0.0)
        hit = _xlane_max(hitv, lane)[0] > 0.5

        @pl.when(jnp.logical_not(hit))
        def _():
            accs = [accb[pl.ds(g * _LANES, _LANES)] for g in range(_GROUPS)]

            @plsc.parallel_loop(0, _SLOT_ROWS, 1, unroll=2,
                                carry=tuple(accs))
            def red(row, acc):
                return tuple(
                    jnp.maximum(a, buf[row, pl.ds(g * _LANES, _LANES)])
                    for g, a in enumerate(acc))

            for g in range(_GROUPS):
                accb[pl.ds(g * _LANES, _LANES)] = red[g]

        @pl.when(hit)
        def _():
            accs = [accb[pl.ds(g * _LANES, _LANES)] for g in range(_GROUPS)]
            fys = [fyb[pl.ds(g * _LANES, _LANES)] for g in range(_GROUPS)]

            @plsc.parallel_loop(0, _SLOT_ROWS, 1,
                                carry=tuple(accs) + tuple(fys))
            def red(row, c):
                out = list(c[:_GROUPS])
                fy = list(c[_GROUPS:])
                rowv = jnp.full((_LANES,), base + row, jnp.int32)
                for g in range(_GROUPS):
                    x = buf[row, pl.ds(g * _LANES, _LANES)]
                    isl = lblv[g] == rowv
                    out[g] = jnp.maximum(out[g], jnp.where(isl, negv, x))
                    fy[g] = jnp.maximum(fy[g], jnp.where(isl, x, negv))
                return tuple(out) + tuple(fy)

            for g in range(_GROUPS):
                accb[pl.ds(g * _LANES, _LANES)] = red[g]
                fyb[pl.ds(g * _LANES, _LANES)] = red[_GROUPS + g]

    def round_body(ri, carry):
        for u in range(_NBUF):
            s = ri * _NBUF + u
            process_slot(s, u)

            @pl.when(s < _NSLOT - _NBUF)
            def _():
                start_slot(s + _NBUF, u)
        return carry

    lax.fori_loop(0, _NROUND, round_body, 0)

    hsum = jnp.zeros((_LANES,), jnp.float32)
    for g in range(_GROUPS):
        fnym = accb[pl.ds(g * _LANES, _LANES)]
        fy = fyb[pl.ds(g * _LANES, _LANES)]
        hsum = hsum + (jnp.maximum(2.0 - fy, 0.0)
                       + jnp.maximum(1.0 + fnym, 0.0))
    sums_v[...] = hsum
    pltpu.sync_copy(sums_v, out_ref.at[wid])


@jax.jit
def _sc_loss(pred_t, lbl):
    mesh = plsc.VectorSubcoreMesh(
        core_axis_name="c", subcore_axis_name="s",
        num_cores=_NUM_CORES, num_subcores=_NUM_SUBCORES)
    part = pl.kernel(
        _sc_body,
        out_type=jax.ShapeDtypeStruct((_NW, _LANES), jnp.float32),
        mesh=mesh,
        compiler_params=pltpu.CompilerParams(use_tc_tiling_on_sc=True),
        scratch_types=(
            [pltpu.VMEM((_WCOLS,), jnp.int32),
             pltpu.VMEM((_WCOLS,), jnp.float32),
             pltpu.VMEM((_WCOLS,), jnp.float32)]
            + [pltpu.VMEM((_SLOT_ROWS, _WCOLS), jnp.float32)] * _NBUF
            + [pltpu.VMEM((_LANES,), jnp.float32),
               pltpu.SemaphoreType.DMA((_NBUF,))]
        ),
    )(pred_t, lbl)
    return jnp.sum(part) * (1.0 / _B)


def kernel(prediction, label):
    return _sc_loss(prediction.T, label.astype(jnp.int32))
